# baseline (speedup 1.0000x reference)
import functools

import jax
import jax.numpy as jnp
from jax.experimental import pallas as pl
from jax.experimental.pallas import tpu as pltpu

_NUM_HIER = 6


def _body(x_ref, hsp_ref, tp_ref, j2h_ref, o_ref):
    j2h = j2h_ref[...]
    J = j2h.shape[0]
    C = hsp_ref.shape[1]
    hsp_g = jnp.zeros((J, C), jnp.float32)
    for h in range(_NUM_HIER):
        hsp_g = jnp.where(j2h == h, hsp_ref[h:h + 1, :], hsp_g)
    tp = tp_ref[...]
    bias = hsp_g[None, :, :] + tp[:, None, :]
    o_ref[...] = x_ref[...] + bias[None, :, :, :]


def kernel(x, hsp_W, tp_W, joint2hier):
    B, N, J, C = x.shape
    j2h = joint2hier.reshape(J, 1)
    grid = (B,)
    return pl.pallas_call(
        _body,
        grid=grid,
        in_specs=[
            pl.BlockSpec((1, N, J, C), lambda i: (i, 0, 0, 0)),
            pl.BlockSpec((_NUM_HIER, C), lambda i: (0, 0)),
            pl.BlockSpec((N, C), lambda i: (0, 0)),
            pl.BlockSpec((J, 1), lambda i: (0, 0)),
        ],
        out_specs=pl.BlockSpec((1, N, J, C), lambda i: (i, 0, 0, 0)),
        out_shape=jax.ShapeDtypeStruct((B, N, J, C), x.dtype),
        compiler_params=pltpu.CompilerParams(
            dimension_semantics=("arbitrary",),
        ),
    )(x, hsp_W, tp_W, j2h)
